# baseline (speedup 1.0000x reference)
import functools

import jax
import jax.numpy as jnp
from jax import lax
from jax.experimental import pallas as pl
from jax.experimental.pallas import tpu as pltpu
from jax.experimental.pallas import tpu_sc as plsc

N = 10000
E = 320000
C = 128
NC = 2
NS = 16
NW = NC * NS
B = 128
CH = 79
CH0 = 131
CH1 = 2 * CH - CH0
EPAD = NW * CH * B
NPAD = 10240
RPT = NPAD // NS
DUMP = 10008

_mesh = plsc.VectorSubcoreMesh(
    core_axis_name="c", subcore_axis_name="s", num_cores=NC, num_subcores=NS)


@functools.partial(
    pl.kernel,
    out_type=jax.ShapeDtypeStruct((NC, NPAD), jnp.float32),
    mesh=_mesh,
    scratch_types=[
        pltpu.VMEM((CH, B), jnp.int32),
        pltpu.VMEM((B,), jnp.int32),
        pltpu.VMEM((B,), jnp.float32),
        pltpu.VMEM((RPT,), jnp.float32),
        pltpu.VMEM_SHARED((NPAD,), jnp.float32),
    ],
)
def _deg_kernel(dst_hbm, out_hbm, dstv, idxb, onesb, zb, degf):
    c = lax.axis_index("c")
    s = lax.axis_index("s")
    w = c * NS + s
    pltpu.sync_copy(dst_hbm.at[w], dstv)
    for k in range(B // 16):
        onesb[pl.ds(k * 16, 16)] = jnp.ones((16,), jnp.float32)
    for k in range(RPT // 16):
        zb[pl.ds(k * 16, 16)] = jnp.zeros((16,), jnp.float32)
    pltpu.sync_copy(zb, degf.at[pl.ds(s * RPT, RPT)])
    plsc.subcore_barrier()

    def chunk(j, carry):
        row = dstv.at[j]
        for k in range(B // 16):
            idxb[pl.ds(k * 16, 16)] = row[pl.ds(k * 16, 16)]
        pltpu.sync_copy(onesb, degf.at[idxb], add=True)
        return carry

    lax.fori_loop(0, CH, chunk, 0)
    plsc.subcore_barrier()
    pltpu.sync_copy(degf.at[pl.ds(s * RPT, RPT)], out_hbm.at[c, pl.ds(s * RPT, RPT)])


@functools.partial(
    pl.kernel,
    out_type=jax.ShapeDtypeStruct((NC, NPAD, C), jnp.float32),
    mesh=_mesh,
    scratch_types=[
        pltpu.VMEM((B,), jnp.int32),
        pltpu.VMEM((B,), jnp.int32),
        pltpu.VMEM((B,), jnp.int32),
        pltpu.VMEM((B,), jnp.int32),
        pltpu.VMEM((B,), jnp.int32),
        pltpu.VMEM((B,), jnp.int32),
        pltpu.VMEM((B, C), jnp.float32),
        pltpu.VMEM((B, C), jnp.float32),
        pltpu.VMEM_SHARED((NPAD, C), jnp.float32),
        pltpu.SemaphoreType.DMA,
        pltpu.SemaphoreType.DMA,
        pltpu.SemaphoreType.DMA,
        pltpu.SemaphoreType.DMA,
        pltpu.SemaphoreType.DMA,
        pltpu.SemaphoreType.DMA,
    ],
)
def _agg_kernel(g_hbm, src_hbm, dst_hbm, zrows_hbm, out_hbm,
                sidx0, sidx1, didx0, didx1, dscat0, dscat1, buf0, buf1, acc,
                gsem0, gsem1, isem0, isem1, ssem0, ssem1):
    c = lax.axis_index("c")
    s = lax.axis_index("s")
    nch = jnp.where(c == 0, CH0, CH1)
    cstart = jnp.where(c == 0, s * CH0, NS * CH0 + s * CH1)
    ebase = cstart * B

    def ids_at(j):
        off = pl.multiple_of(ebase + j * B, B)
        return src_hbm.at[pl.ds(off, B)], dst_hbm.at[pl.ds(off, B)]

    def fetch_ids(j, sidx, didx, isem):
        sh, dh = ids_at(j)
        pltpu.async_copy(sh, sidx, isem)
        pltpu.async_copy(dh, didx, isem)

    def wait_ids(j, sidx, didx, isem):
        sh, dh = ids_at(j)
        pltpu.make_async_copy(sh, sidx, isem).wait()
        pltpu.make_async_copy(dh, didx, isem).wait()

    def vcopy(src_ref, dst_ref):
        for k in range(B // 16):
            dst_ref[pl.ds(k * 16, 16)] = src_ref[pl.ds(k * 16, 16)]

    base = s * RPT
    for k in range(RPT // B):
        pltpu.async_copy(zrows_hbm.at[pl.ds(0, B)],
                        acc.at[pl.ds(base + k * B, B)], gsem0)
    for k in range(RPT // B):
        pltpu.make_async_copy(zrows_hbm.at[pl.ds(0, B)],
                              acc.at[pl.ds(base + k * B, B)], gsem0).wait()
    plsc.subcore_barrier()

    fetch_ids(0, sidx0, didx0, isem0)
    wait_ids(0, sidx0, didx0, isem0)
    fetch_ids(jnp.minimum(1, nch - 1), sidx1, didx1, isem1)
    pltpu.async_copy(g_hbm.at[sidx0], buf0, gsem0)

    def sub(j, sa, sb, da, db, dsa, ba, bb, ga, gb, ia, ib, sca, scb, first):
        wait_ids(jnp.minimum(j + 1, nch - 1), sb, db, ib)
        pltpu.make_async_copy(g_hbm.at[sa], ba, ga).wait()
        pltpu.async_copy(g_hbm.at[sb], bb, gb)
        pltpu.sync_copy(ba, acc.at[da], add=True)
        fetch_ids(jnp.minimum(j + 2, nch - 1), sa, da, ia)

    sub(0, sidx0, sidx1, didx0, didx1, dscat0, buf0, buf1,
        gsem0, gsem1, isem0, isem1, ssem0, ssem1, True)

    def body2(i, carry):
        j = 2 * i + 1
        sub(j, sidx1, sidx0, didx1, didx0, dscat1, buf1, buf0,
            gsem1, gsem0, isem1, isem0, ssem1, ssem0, False)
        sub(j + 1, sidx0, sidx1, didx0, didx1, dscat0, buf0, buf1,
            gsem0, gsem1, isem0, isem1, ssem0, ssem1, False)
        return carry

    lax.fori_loop(0, (nch - 1) // 2, body2, 0)
    pltpu.make_async_copy(g_hbm.at[sidx1], buf1, gsem1).wait()
    wait_ids(nch - 1, sidx0, didx0, isem0)
    plsc.subcore_barrier()
    pltpu.sync_copy(acc.at[pl.ds(base, RPT)], out_hbm.at[c, pl.ds(base, RPT)])


def _lin_body(x_ref, w_ref, degT_ref, g_ref):
    h = jnp.dot(x_ref[...], w_ref[...], preferred_element_type=jnp.float32)
    deg = degT_ref[:, 0:1] + degT_ref[:, 1:2] + 1.0
    g_ref[...] = h * lax.rsqrt(deg)


def _fin_body(p0_ref, p1_ref, g_ref, degT_ref, b_ref, o_ref):
    deg = degT_ref[:, 0:1] + degT_ref[:, 1:2] + 1.0
    dis = lax.rsqrt(deg)
    acc = p0_ref[0] + p1_ref[0] + g_ref[...]
    o_ref[...] = jnp.maximum(dis * acc + b_ref[...], 0.0)


_RB = 1000


def kernel(x, edge_index, W, b):
    src = edge_index[0].astype(jnp.int32)
    dst = edge_index[1].astype(jnp.int32)
    pad = EPAD - E
    src1 = jnp.concatenate([src, jnp.zeros((pad,), jnp.int32)])
    dst1 = jnp.concatenate([dst, jnp.full((pad,), DUMP, jnp.int32)])
    dst3 = dst1.reshape(NW, CH, B)

    zrows = jnp.zeros((B, C), jnp.float32)

    degw = _deg_kernel(dst3)
    degT = degw[:, :N].T

    g = pl.pallas_call(
        _lin_body,
        grid=(N // _RB,),
        in_specs=[
            pl.BlockSpec((_RB, C), lambda i: (i, 0)),
            pl.BlockSpec((C, C), lambda i: (0, 0)),
            pl.BlockSpec((_RB, 2), lambda i: (i, 0)),
        ],
        out_specs=pl.BlockSpec((_RB, C), lambda i: (i, 0)),
        out_shape=jax.ShapeDtypeStruct((N, C), jnp.float32),
    )(x, W, degT)

    parts = _agg_kernel(g, src1, dst1, zrows)

    out = pl.pallas_call(
        _fin_body,
        grid=(N // _RB,),
        in_specs=[
            pl.BlockSpec((1, _RB, C), lambda i: (0, i, 0)),
            pl.BlockSpec((1, _RB, C), lambda i: (1, i, 0)),
            pl.BlockSpec((_RB, C), lambda i: (i, 0)),
            pl.BlockSpec((_RB, 2), lambda i: (i, 0)),
            pl.BlockSpec((1, C), lambda i: (0, 0)),
        ],
        out_specs=pl.BlockSpec((_RB, C), lambda i: (i, 0)),
        out_shape=jax.ShapeDtypeStruct((N, C), jnp.float32),
    )(parts, parts, g, degT, b.reshape(1, C))
    return out


# speedup vs baseline: 1.1749x; 1.1749x over previous
import functools

import jax
import jax.numpy as jnp
from jax import lax
from jax.experimental import pallas as pl
from jax.experimental.pallas import tpu as pltpu
from jax.experimental.pallas import tpu_sc as plsc

N = 10000
E = 320000
C = 128
NC = 2
NS = 16
NW = NC * NS
B = 128
EPT = E // NW
NPAD = 10240
RPT = NPAD // NS

S0_BIG, S0_SMALL, S0_NBIG = 121, 119, 10
S1_BIG, S1_SMALL, S1_NBIG = 37, 35, 8
S0_TOT = S0_NBIG * S0_BIG + (NS - S0_NBIG) * S0_SMALL

_mesh = plsc.VectorSubcoreMesh(
    core_axis_name="c", subcore_axis_name="s", num_cores=NC, num_subcores=NS)


@functools.partial(
    pl.kernel,
    out_type=jax.ShapeDtypeStruct((NC, NPAD), jnp.float32),
    mesh=_mesh,
    scratch_types=[
        pltpu.VMEM((EPT,), jnp.int32),
        pltpu.VMEM((B,), jnp.int32),
        pltpu.VMEM((B,), jnp.float32),
        pltpu.VMEM((RPT,), jnp.float32),
        pltpu.VMEM_SHARED((NPAD,), jnp.float32),
    ],
)
def _deg_kernel(dst_hbm, out_hbm, dstv, idxb, onesb, zb, degf):
    c = lax.axis_index("c")
    s = lax.axis_index("s")
    w = c * NS + s
    pltpu.sync_copy(dst_hbm.at[pl.ds(pl.multiple_of(w * EPT, 8), EPT)], dstv)
    for k in range(B // 16):
        onesb[pl.ds(k * 16, 16)] = jnp.ones((16,), jnp.float32)
    for k in range(RPT // 16):
        zb[pl.ds(k * 16, 16)] = jnp.zeros((16,), jnp.float32)
    pltpu.sync_copy(zb, degf.at[pl.ds(s * RPT, RPT)])
    plsc.subcore_barrier()

    def chunk(j, carry):
        base = j * B
        for k in range(B // 16):
            idxb[pl.ds(k * 16, 16)] = dstv[pl.ds(base + k * 16, 16)]
        pltpu.sync_copy(onesb, degf.at[idxb], add=True)
        return carry

    lax.fori_loop(0, EPT // B, chunk, 0)
    for k in range(B // 16):
        idxb[pl.ds(k * 16, 16)] = jnp.full((16,), N + 64, jnp.int32)
    for k in range((EPT % B) // 16):
        idxb[pl.ds(k * 16, 16)] = dstv[pl.ds((EPT // B) * B + k * 16, 16)]
    pltpu.sync_copy(onesb, degf.at[idxb], add=True)
    plsc.subcore_barrier()
    pltpu.sync_copy(degf.at[pl.ds(s * RPT, RPT)], out_hbm.at[c, pl.ds(s * RPT, RPT)])


@functools.partial(
    pl.kernel,
    out_type=jax.ShapeDtypeStruct((NC, NPAD, C), jnp.float32),
    mesh=_mesh,
    scratch_types=[
        pltpu.VMEM((B,), jnp.int32),
        pltpu.VMEM((B,), jnp.int32),
        pltpu.VMEM((B,), jnp.int32),
        pltpu.VMEM((B,), jnp.int32),
        pltpu.VMEM((B,), jnp.int32),
        pltpu.VMEM((B,), jnp.int32),
        pltpu.VMEM((B, C), jnp.float32),
        pltpu.VMEM((B, C), jnp.float32),
        pltpu.VMEM_SHARED((NPAD, C), jnp.float32),
        pltpu.SemaphoreType.DMA,
        pltpu.SemaphoreType.DMA,
        pltpu.SemaphoreType.DMA,
        pltpu.SemaphoreType.DMA,
        pltpu.SemaphoreType.DMA,
        pltpu.SemaphoreType.DMA,
    ],
)
def _agg_kernel(g_hbm, src_hbm, dst_hbm, zrows_hbm, out_hbm,
                sidx0, sidx1, didx0, didx1, dscat0, dscat1, buf0, buf1, acc,
                gsem0, gsem1, isem0, isem1, ssem0, ssem1):
    c = lax.axis_index("c")
    s = lax.axis_index("s")
    nch0 = jnp.where(s < S0_NBIG, S0_BIG, S0_SMALL)
    st0 = (S0_BIG * jnp.minimum(s, S0_NBIG)
           + S0_SMALL * jnp.maximum(s - S0_NBIG, 0))
    nch1 = jnp.where(s < S1_NBIG, S1_BIG, S1_SMALL)
    st1 = (S0_TOT + S1_BIG * jnp.minimum(s, S1_NBIG)
           + S1_SMALL * jnp.maximum(s - S1_NBIG, 0))
    nch = jnp.where(c == 0, nch0, nch1)
    cstart = jnp.where(c == 0, st0, st1)
    ebase = cstart * B

    def ids_at(j):
        off = pl.multiple_of(ebase + j * B, B)
        return src_hbm.at[pl.ds(off, B)], dst_hbm.at[pl.ds(off, B)]

    def fetch_ids(j, sidx, didx, isem):
        sh, dh = ids_at(j)
        pltpu.async_copy(sh, sidx, isem)
        pltpu.async_copy(dh, didx, isem)

    def wait_ids(j, sidx, didx, isem):
        sh, dh = ids_at(j)
        pltpu.make_async_copy(sh, sidx, isem).wait()
        pltpu.make_async_copy(dh, didx, isem).wait()

    def vcopy(src_ref, dst_ref):
        for k in range(B // 16):
            dst_ref[pl.ds(k * 16, 16)] = src_ref[pl.ds(k * 16, 16)]

    base = s * RPT
    for k in range(RPT // B):
        pltpu.async_copy(zrows_hbm.at[pl.ds(0, B)],
                        acc.at[pl.ds(base + k * B, B)], gsem0)
    for k in range(RPT // B):
        pltpu.make_async_copy(zrows_hbm.at[pl.ds(0, B)],
                              acc.at[pl.ds(base + k * B, B)], gsem0).wait()
    plsc.subcore_barrier()

    fetch_ids(0, sidx0, didx0, isem0)
    wait_ids(0, sidx0, didx0, isem0)
    fetch_ids(jnp.minimum(1, nch - 1), sidx1, didx1, isem1)
    pltpu.async_copy(g_hbm.at[sidx0], buf0, gsem0)

    def sub(j, sa, sb, da, db, dsa, ba, bb, ga, gb, ia, ib, sca, scb, first):
        wait_ids(jnp.minimum(j + 1, nch - 1), sb, db, ib)
        pltpu.make_async_copy(g_hbm.at[sa], ba, ga).wait()
        pltpu.async_copy(g_hbm.at[sb], bb, gb)
        pltpu.sync_copy(ba, acc.at[da], add=True)
        fetch_ids(jnp.minimum(j + 2, nch - 1), sa, da, ia)

    sub(0, sidx0, sidx1, didx0, didx1, dscat0, buf0, buf1,
        gsem0, gsem1, isem0, isem1, ssem0, ssem1, True)

    def body2(i, carry):
        j = 2 * i + 1
        sub(j, sidx1, sidx0, didx1, didx0, dscat1, buf1, buf0,
            gsem1, gsem0, isem1, isem0, ssem1, ssem0, False)
        sub(j + 1, sidx0, sidx1, didx0, didx1, dscat0, buf0, buf1,
            gsem0, gsem1, isem0, isem1, ssem0, ssem1, False)
        return carry

    lax.fori_loop(0, (nch - 1) // 2, body2, 0)
    pltpu.make_async_copy(g_hbm.at[sidx1], buf1, gsem1).wait()
    wait_ids(nch - 1, sidx0, didx0, isem0)
    plsc.subcore_barrier()
    pltpu.sync_copy(acc.at[pl.ds(base, RPT)], out_hbm.at[c, pl.ds(base, RPT)])


def _mm_body(x_ref, w_ref, h_ref):
    h_ref[...] = jnp.dot(x_ref[...], w_ref[...],
                         precision=lax.Precision.HIGHEST,
                         preferred_element_type=jnp.float32)


def _scale_body(h_ref, degT_ref, g_ref):
    deg = degT_ref[:, 0:1] + degT_ref[:, 1:2] + 1.0
    g_ref[...] = h_ref[...] * lax.rsqrt(deg)


def _fin_body(p0_ref, p1_ref, g_ref, degT_ref, b_ref, o_ref):
    deg = degT_ref[:, 0:1] + degT_ref[:, 1:2] + 1.0
    dis = lax.rsqrt(deg)
    acc = p0_ref[0] + p1_ref[0] + g_ref[...]
    o_ref[...] = jnp.maximum(dis * acc + b_ref[...], 0.0)


_RB = 1000


def kernel(x, edge_index, W, b):
    src1 = edge_index[0].astype(jnp.int32)
    dst1 = edge_index[1].astype(jnp.int32)
    zrows = jnp.zeros((B, C), jnp.float32)

    h = pl.pallas_call(
        _mm_body,
        grid=(N // _RB,),
        in_specs=[
            pl.BlockSpec((_RB, C), lambda i: (i, 0)),
            pl.BlockSpec((C, C), lambda i: (0, 0)),
        ],
        out_specs=pl.BlockSpec((_RB, C), lambda i: (i, 0)),
        out_shape=jax.ShapeDtypeStruct((N, C), jnp.float32),
    )(x, W)

    degw = _deg_kernel(dst1)
    degT = degw[:, :N].T

    g = pl.pallas_call(
        _scale_body,
        grid=(N // _RB,),
        in_specs=[
            pl.BlockSpec((_RB, C), lambda i: (i, 0)),
            pl.BlockSpec((_RB, 2), lambda i: (i, 0)),
        ],
        out_specs=pl.BlockSpec((_RB, C), lambda i: (i, 0)),
        out_shape=jax.ShapeDtypeStruct((N, C), jnp.float32),
    )(h, degT)

    parts = _agg_kernel(g, src1, dst1, zrows)

    out = pl.pallas_call(
        _fin_body,
        grid=(N // _RB,),
        in_specs=[
            pl.BlockSpec((1, _RB, C), lambda i: (0, i, 0)),
            pl.BlockSpec((1, _RB, C), lambda i: (1, i, 0)),
            pl.BlockSpec((_RB, C), lambda i: (i, 0)),
            pl.BlockSpec((_RB, 2), lambda i: (i, 0)),
            pl.BlockSpec((1, C), lambda i: (0, 0)),
        ],
        out_specs=pl.BlockSpec((_RB, C), lambda i: (i, 0)),
        out_shape=jax.ShapeDtypeStruct((N, C), jnp.float32),
    )(parts, parts, g, degT, b.reshape(1, C))
    return out
